# baseline (speedup 1.0000x reference)
import functools

import jax
import jax.numpy as jnp
from jax import lax
from jax.experimental import pallas as pl
from jax.experimental.pallas import tpu as pltpu
from jax.experimental.pallas import tpu_sc as plsc

_BLK = 102


def _fuse_body(fl_ref, cl_ref, ft_ref, ct_ref, w_ref, b_ref, out_ref,
               idx_ref):
    nf, c = ft_ref.shape
    ncol = ct_ref.shape[0]
    w1 = w_ref[:, :c]
    w2 = w_ref[:, c:]
    f = lax.dot_general(ft_ref[...], w1, (((1,), (1,)), ((), ())),
                        preferred_element_type=jnp.float32)
    g = lax.dot_general(ct_ref[...], w2, (((1,), (1,)), ((), ())),
                        preferred_element_type=jnp.float32) + b_ref[...]
    for col in range(ncol):
        out_ref[pl.ds(col * _BLK, nf), :] = f + g[col:col + 1, :]
    idx_ref[...] = cl_ref[...] * _BLK + fl_ref[...]


def _fused_table(flower_label, color_label, flower_table, color_table, W, b):
    nf, c = flower_table.shape
    nc = color_table.shape[0]
    batch = flower_label.shape[0]
    return pl.pallas_call(
        _fuse_body,
        out_shape=(jax.ShapeDtypeStruct((nc * _BLK, c), jnp.float32),
                   jax.ShapeDtypeStruct((batch,), jnp.int32)),
    )(flower_label, color_label, flower_table, color_table, W,
      b.reshape(1, c))


def _sc_lookup(table, idx):
    batch = idx.shape[0]
    c = table.shape[1]
    info = plsc.get_sparse_core_info()
    nw = info.num_cores * info.num_subcores
    bpw = batch // nw
    ch = 64
    nch = bpw // ch
    nbuf = 7

    mesh = plsc.VectorSubcoreMesh(core_axis_name="c", subcore_axis_name="s")

    @functools.partial(
        pl.kernel,
        mesh=mesh,
        out_type=jax.ShapeDtypeStruct((batch, c), jnp.float32),
        compiler_params=pltpu.CompilerParams(use_tc_tiling_on_sc=False),
        scratch_types=[
            pltpu.VMEM((bpw,), jnp.int32),
        ] + [pltpu.VMEM((ch, c), jnp.float32)] * nbuf
          + [pltpu.SemaphoreType.DMA] * (2 * nbuf),
    )
    def k(idx_hbm, tab_hbm, out_hbm, idx2, *rest):
        bufs = rest[:nbuf]
        gsem = rest[nbuf:2 * nbuf]
        wsem = rest[2 * nbuf:]
        wid = lax.axis_index("s") * info.num_cores + lax.axis_index("c")
        base = wid * bpw
        pltpu.sync_copy(idx_hbm.at[pl.ds(base, bpw)], idx2)
        gcp = [None] * nch
        wcp = [None] * nch
        for j in range(min(nbuf, nch)):
            gcp[j] = pltpu.async_copy(
                tab_hbm.at[idx2.at[pl.ds(j * ch, ch)]], bufs[j % nbuf],
                gsem[j % nbuf])
        for j in range(nch):
            gcp[j].wait()
            wcp[j] = pltpu.async_copy(
                bufs[j % nbuf], out_hbm.at[pl.ds(base + j * ch, ch)],
                wsem[j % nbuf])
            nxt = j + nbuf
            if nxt < nch:
                wcp[j].wait()
                wcp[j] = None
                gcp[nxt] = pltpu.async_copy(
                    tab_hbm.at[idx2.at[pl.ds(nxt * ch, ch)]],
                    bufs[nxt % nbuf], gsem[nxt % nbuf])
        for j in range(nch):
            if wcp[j] is not None:
                wcp[j].wait()

    return k(idx, table)


def kernel(flower_label, color_label, flower_table, color_table, W, b):
    tab, idx = _fused_table(flower_label.astype(jnp.int32),
                            color_label.astype(jnp.int32),
                            flower_table, color_table, W, b)
    return _sc_lookup(tab, idx)


# speedup vs baseline: 1.5185x; 1.5185x over previous
import functools

import jax
import jax.numpy as jnp
from jax import lax
from jax.experimental import pallas as pl
from jax.experimental.pallas import tpu as pltpu
from jax.experimental.pallas import tpu_sc as plsc

_BLK = 102


def _fuse_body(fl_ref, cl_ref, ft_ref, ct_ref, w_ref, b_ref, out_ref,
               idx_ref):
    nf, c = ft_ref.shape
    ncol = ct_ref.shape[0]
    half = c // 2
    w1 = w_ref[:, :c]
    w2 = w_ref[:, c:]
    f = lax.dot_general(ft_ref[...], w1, (((1,), (1,)), ((), ())),
                        preferred_element_type=jnp.float32)
    g = lax.dot_general(ct_ref[...], w2, (((1,), (1,)), ((), ())),
                        preferred_element_type=jnp.float32) + b_ref[...]
    for col in range(ncol):
        fg = f + g[col:col + 1, :]
        out_ref[pl.ds(col * _BLK, nf), 0, :] = fg[:, :half]
        out_ref[pl.ds(col * _BLK, nf), 1, :] = fg[:, half:]
    idx_ref[...] = cl_ref[...] * _BLK + fl_ref[...]


def _fused_table(flower_label, color_label, flower_table, color_table, W, b):
    nf, c = flower_table.shape
    nc = color_table.shape[0]
    batch = flower_label.shape[0]
    return pl.pallas_call(
        _fuse_body,
        out_shape=(jax.ShapeDtypeStruct((nc * _BLK, 2, c // 2), jnp.float32),
                   jax.ShapeDtypeStruct((batch,), jnp.int32)),
    )(flower_label, color_label, flower_table, color_table, W,
      b.reshape(1, c))


def _sc_lookup(table, idx, c):
    batch = idx.shape[0]
    half = c // 2
    info = plsc.get_sparse_core_info()
    nw = info.num_cores * info.num_subcores
    bpw = batch // nw
    ch = 64
    nch = bpw // ch
    nbuf = 7

    mesh = plsc.VectorSubcoreMesh(core_axis_name="c", subcore_axis_name="s")

    @functools.partial(
        pl.kernel,
        mesh=mesh,
        out_type=jax.ShapeDtypeStruct((batch, c), jnp.float32),
        scratch_types=[
            pltpu.VMEM((bpw,), jnp.int32),
        ] + [pltpu.VMEM((ch, 2, half), jnp.float32)] * nbuf
          + [pltpu.SemaphoreType.DMA] * (3 * nbuf),
    )
    def k(idx_hbm, tab_hbm, out_hbm, idx2, *rest):
        bufs = rest[:nbuf]
        gsem = rest[nbuf:2 * nbuf]
        wsem0 = rest[2 * nbuf:3 * nbuf]
        wsem1 = rest[3 * nbuf:]
        wid = lax.axis_index("s") * info.num_cores + lax.axis_index("c")
        base = wid * bpw
        pltpu.sync_copy(idx_hbm.at[pl.ds(base, bpw)], idx2)
        gcp = [None] * nch
        wcp = [None] * nch
        for j in range(min(nbuf, nch)):
            gcp[j] = pltpu.async_copy(
                tab_hbm.at[idx2.at[pl.ds(j * ch, ch)]], bufs[j % nbuf],
                gsem[j % nbuf])
        for j in range(nch):
            gcp[j].wait()
            rows = pl.ds(base + j * ch, ch)
            wcp[j] = (
                pltpu.async_copy(bufs[j % nbuf].at[:, 0, :],
                                 out_hbm.at[rows, pl.ds(0, half)],
                                 wsem0[j % nbuf]),
                pltpu.async_copy(bufs[j % nbuf].at[:, 1, :],
                                 out_hbm.at[rows, pl.ds(half, half)],
                                 wsem1[j % nbuf]),
            )
            nxt = j + nbuf
            if nxt < nch:
                wcp[j][0].wait()
                wcp[j][1].wait()
                wcp[j] = None
                gcp[nxt] = pltpu.async_copy(
                    tab_hbm.at[idx2.at[pl.ds(nxt * ch, ch)]],
                    bufs[nxt % nbuf], gsem[nxt % nbuf])
        for j in range(nch):
            if wcp[j] is not None:
                wcp[j][0].wait()
                wcp[j][1].wait()

    return k(idx, table)


def kernel(flower_label, color_label, flower_table, color_table, W, b):
    tab, idx = _fused_table(flower_label.astype(jnp.int32),
                            color_label.astype(jnp.int32),
                            flower_table, color_table, W, b)
    return _sc_lookup(tab, idx, flower_table.shape[1])


# speedup vs baseline: 1.5876x; 1.0455x over previous
import functools

import jax
import jax.numpy as jnp
from jax import lax
from jax.experimental import pallas as pl
from jax.experimental.pallas import tpu as pltpu
from jax.experimental.pallas import tpu_sc as plsc

_BLK = 102


def _fuse_body(fl_ref, cl_ref, ft_ref, ct_ref, w_ref, b_ref, out_ref,
               idx_ref):
    nf, c = ft_ref.shape
    ncol = ct_ref.shape[0]
    w1 = w_ref[:, :c]
    w2 = w_ref[:, c:]
    f = lax.dot_general(ft_ref[...], w1, (((1,), (1,)), ((), ())),
                        preferred_element_type=jnp.float32)
    g = lax.dot_general(ct_ref[...], w2, (((1,), (1,)), ((), ())),
                        preferred_element_type=jnp.float32) + b_ref[...]
    for col in range(ncol):
        out_ref[pl.ds(col * _BLK, nf), :] = f + g[col:col + 1, :]
    idx_ref[...] = cl_ref[...] * _BLK + fl_ref[...]


def _fused_table(flower_label, color_label, flower_table, color_table, W, b):
    nf, c = flower_table.shape
    nc = color_table.shape[0]
    batch = flower_label.shape[0]
    return pl.pallas_call(
        _fuse_body,
        out_shape=(jax.ShapeDtypeStruct((nc * _BLK, c), jnp.float32),
                   jax.ShapeDtypeStruct((batch,), jnp.int32)),
    )(flower_label, color_label, flower_table, color_table, W,
      b.reshape(1, c))


def _sc_lookup(table, idx):
    batch = idx.shape[0]
    c = table.shape[1]
    info = plsc.get_sparse_core_info()
    nw = info.num_cores * info.num_subcores
    bpw = batch // nw
    ch = 128
    nch = bpw // ch
    nbuf = 3

    mesh = plsc.VectorSubcoreMesh(core_axis_name="c", subcore_axis_name="s")

    @functools.partial(
        pl.kernel,
        mesh=mesh,
        out_type=jax.ShapeDtypeStruct((batch, c), jnp.float32),
        scratch_types=[
            pltpu.VMEM((bpw,), jnp.int32),
        ] + [pltpu.VMEM((ch, c), jnp.float32)] * nbuf
          + [pltpu.SemaphoreType.DMA] * (2 * nbuf),
    )
    def k(idx_hbm, tab_hbm, out_hbm, idx2, *rest):
        bufs = rest[:nbuf]
        gsem = rest[nbuf:2 * nbuf]
        wsem = rest[2 * nbuf:]
        wid = lax.axis_index("s") * info.num_cores + lax.axis_index("c")
        base = wid * bpw
        pltpu.sync_copy(idx_hbm.at[pl.ds(base, bpw)], idx2)
        gcp = [None] * nch
        wcp = [None] * nch
        for j in range(min(nbuf, nch)):
            gcp[j] = pltpu.async_copy(
                tab_hbm.at[idx2.at[pl.ds(j * ch, ch)]], bufs[j % nbuf],
                gsem[j % nbuf])
        for j in range(nch):
            gcp[j].wait()
            wcp[j] = pltpu.async_copy(
                bufs[j % nbuf], out_hbm.at[pl.ds(base + j * ch, ch)],
                wsem[j % nbuf])
            nxt = j + nbuf
            if nxt < nch:
                wcp[j].wait()
                wcp[j] = None
                gcp[nxt] = pltpu.async_copy(
                    tab_hbm.at[idx2.at[pl.ds(nxt * ch, ch)]],
                    bufs[nxt % nbuf], gsem[nxt % nbuf])
        for j in range(nch):
            if wcp[j] is not None:
                wcp[j].wait()

    return k(idx, table)


def kernel(flower_label, color_label, flower_table, color_table, W, b):
    tab, idx = _fused_table(flower_label.astype(jnp.int32),
                            color_label.astype(jnp.int32),
                            flower_table, color_table, W, b)
    return _sc_lookup(tab, idx)


# speedup vs baseline: 1.6008x; 1.0083x over previous
import functools

import jax
import jax.numpy as jnp
from jax import lax
from jax.experimental import pallas as pl
from jax.experimental.pallas import tpu as pltpu
from jax.experimental.pallas import tpu_sc as plsc

_BLK = 102


def _fuse_body(fl_ref, cl_ref, ft_ref, ct_ref, w_ref, b_ref, out_ref,
               idx_ref):
    nf, c = ft_ref.shape
    ncol = ct_ref.shape[0]
    w1 = w_ref[:, :c]
    w2 = w_ref[:, c:]
    f = lax.dot_general(ft_ref[...], w1, (((1,), (1,)), ((), ())),
                        preferred_element_type=jnp.float32)
    g = lax.dot_general(ct_ref[...], w2, (((1,), (1,)), ((), ())),
                        preferred_element_type=jnp.float32) + b_ref[...]
    for col in range(ncol):
        out_ref[pl.ds(col * _BLK, nf), :] = f + g[col:col + 1, :]
    idx_ref[...] = cl_ref[...] * _BLK + fl_ref[...]


def _fused_table(flower_label, color_label, flower_table, color_table, W, b):
    nf, c = flower_table.shape
    nc = color_table.shape[0]
    batch = flower_label.shape[0]
    return pl.pallas_call(
        _fuse_body,
        out_shape=(jax.ShapeDtypeStruct((nc * _BLK, c), jnp.float32),
                   jax.ShapeDtypeStruct((batch,), jnp.int32)),
    )(flower_label, color_label, flower_table, color_table, W,
      b.reshape(1, c))


def _sc_lookup(table, idx):
    batch = idx.shape[0]
    c = table.shape[1]
    info = plsc.get_sparse_core_info()
    nw = info.num_cores * info.num_subcores
    bpw = batch // nw
    ch = 64
    nch = bpw // ch
    nbuf = 7

    mesh = plsc.VectorSubcoreMesh(core_axis_name="c", subcore_axis_name="s")

    @functools.partial(
        pl.kernel,
        mesh=mesh,
        out_type=jax.ShapeDtypeStruct((batch, c), jnp.float32),
        scratch_types=[
            pltpu.VMEM((bpw,), jnp.int32),
        ] + [pltpu.VMEM((ch, c), jnp.float32)] * nbuf
          + [pltpu.SemaphoreType.DMA] * (2 * nbuf),
    )
    def k(idx_hbm, tab_hbm, out_hbm, idx2, *rest):
        bufs = rest[:nbuf]
        gsem = rest[nbuf:2 * nbuf]
        wsem = rest[2 * nbuf:]
        wid = lax.axis_index("s") * info.num_cores + lax.axis_index("c")
        base = wid * bpw
        pltpu.sync_copy(idx_hbm.at[pl.ds(base, bpw)], idx2)
        gcp = [None] * nch
        wcp = [None] * nch
        for j in range(min(nbuf, nch)):
            gcp[j] = pltpu.async_copy(
                tab_hbm.at[idx2.at[pl.ds(j * ch, ch)]], bufs[j % nbuf],
                gsem[j % nbuf])
        for j in range(nch):
            gcp[j].wait()
            wcp[j] = pltpu.async_copy(
                bufs[j % nbuf], out_hbm.at[pl.ds(base + j * ch, ch)],
                wsem[j % nbuf])
            nxt = j + nbuf
            if nxt < nch:
                wcp[j].wait()
                wcp[j] = None
                gcp[nxt] = pltpu.async_copy(
                    tab_hbm.at[idx2.at[pl.ds(nxt * ch, ch)]],
                    bufs[nxt % nbuf], gsem[nxt % nbuf])
        for j in range(nch):
            if wcp[j] is not None:
                wcp[j].wait()

    return k(idx, table)


def kernel(flower_label, color_label, flower_table, color_table, W, b):
    tab, idx = _fused_table(flower_label.astype(jnp.int32),
                            color_label.astype(jnp.int32),
                            flower_table, color_table, W, b)
    return _sc_lookup(tab, idx)
